# baseline (speedup 1.0000x reference)
import functools

import jax
import jax.numpy as jnp
from jax import lax
from jax.experimental import pallas as pl
from jax.experimental.pallas import tpu as pltpu
from jax.experimental.pallas import tpu_sc as plsc

_DIM = 64
_SCALE = 8.0
_LANES = 16
_CHUNK = 256


def _widen_table_t(table_t):
    v = table_t.shape[1]
    block = 2048
    nfull = v // block
    tail = v - nfull * block
    tail_main = (tail // 128) * 128
    tail_rest = tail - tail_main
    grid = nfull + (1 if tail else 0)
    tail_cols = lax.slice(table_t, (0, v - tail_rest), (_DIM, v))

    def body(x_hbm, t_ref, o_ref, xbuf, sem):
        i = pl.program_id(0)

        @pl.when(i < nfull)
        def _full():
            cp = pltpu.make_async_copy(
                x_hbm.at[:, pl.ds(i * block, block)], xbuf, sem
            )
            cp.start()
            cp.wait()

        if tail:

            @pl.when(i == nfull)
            def _tail():
                cp = pltpu.make_async_copy(
                    x_hbm.at[:, pl.ds(nfull * block, tail_main)],
                    xbuf.at[:, pl.ds(0, tail_main)],
                    sem,
                )
                cp.start()
                cp.wait()
                xbuf[:, pl.ds(tail_main, tail_rest)] = t_ref[...]

        xt = jnp.transpose(xbuf[...])
        o_ref[...] = jnp.concatenate([xt, xt], axis=1)

    return pl.pallas_call(
        body,
        grid=(grid,),
        in_specs=[
            pl.BlockSpec(memory_space=pl.ANY),
            pl.BlockSpec((_DIM, tail_rest), lambda i: (0, 0)),
        ],
        out_specs=pl.BlockSpec((block, 2 * _DIM), lambda i: (i, 0)),
        out_shape=jax.ShapeDtypeStruct((grid * block, 2 * _DIM), jnp.float32),
        scratch_shapes=[
            pltpu.VMEM((_DIM, block), jnp.float32),
            pltpu.SemaphoreType.DMA,
        ],
    )(table_t, tail_cols)


def _transpose_out(packed, b, h):
    bb = 128
    rows_per_blk = bb * h * _DIM // 128
    j = h // 2

    def body(x_ref, o_ref):
        x = x_ref[...]
        left = x[:, :_DIM].reshape(bb, j, _DIM)
        right = x[:, _DIM:].reshape(bb, j, _DIM)
        ye = jnp.transpose(left, (1, 2, 0))
        yo = jnp.transpose(right, (1, 2, 0))
        o_ref[...] = jnp.stack([ye, yo], axis=1).reshape(h, _DIM, bb)

    return pl.pallas_call(
        body,
        grid=(b // bb,),
        in_specs=[pl.BlockSpec((rows_per_blk, 128), lambda i: (i, 0))],
        out_specs=pl.BlockSpec((h, _DIM, bb), lambda i: (0, 0, i)),
        out_shape=jax.ShapeDtypeStruct((h, _DIM, b), jnp.float32),
    )(packed)


@functools.lru_cache(maxsize=None)
def _make_gather_kernel(batch_flat: int):
    info = plsc.get_sparse_core_info()
    nc, ns = info.num_cores, info.num_subcores
    nw = nc * ns
    assert batch_flat % nw == 0
    b_per_w = batch_flat // nw
    chunk = _CHUNK
    assert b_per_w % (2 * chunk) == 0
    n_chunks = b_per_w // chunk
    vecs_per_row = _DIM // _LANES

    mesh = plsc.VectorSubcoreMesh(core_axis_name="c", subcore_axis_name="s")

    @functools.partial(
        pl.kernel,
        mesh=mesh,
        out_type=jax.ShapeDtypeStruct((batch_flat * _DIM // 128, 128), jnp.float32),
        scratch_types=[
            pltpu.VMEM((b_per_w,), jnp.int32),
            pltpu.VMEM((2, chunk, 128), jnp.float32),
            pltpu.VMEM((chunk // 2, 128), jnp.float32),
            pltpu.SemaphoreType.DMA,
            pltpu.SemaphoreType.DMA,
        ],
    )
    def k(ids_hbm, table_hbm, out_hbm, idx_v, rows_v, pack_v, gsem0, gsem1):
        wid = lax.axis_index("s") * nc + lax.axis_index("c")
        base = pl.multiple_of(wid * b_per_w, b_per_w)
        gsems = (gsem0, gsem1)

        pltpu.sync_copy(ids_hbm.at[pl.ds(base, b_per_w)], idx_v)
        pltpu.async_copy(
            table_hbm.at[idx_v.at[pl.ds(0, chunk)]], rows_v.at[0], gsem0
        )

        def pair_body(p, carry):
            for b in range(2):
                g = 2 * p + b
                nb = 1 - b

                @pl.when(g + 1 < n_chunks)
                def _start_next():
                    pltpu.async_copy(
                        table_hbm.at[
                            idx_v.at[
                                pl.ds(pl.multiple_of((g + 1) * chunk, chunk), chunk)
                            ]
                        ],
                        rows_v.at[nb],
                        gsems[nb],
                    )

                pltpu.make_async_copy(
                    table_hbm.at[idx_v.at[pl.ds(0, chunk)]],
                    rows_v.at[b],
                    gsems[b],
                ).wait()

                gbuf = rows_v.at[b]

                @plsc.parallel_loop(0, chunk, unroll=4)
                def _scale_row(r):
                    half = (r % 2) * _DIM
                    for v in range(vecs_per_row):
                        pack_v[r // 2, pl.ds(half + v * _LANES, _LANES)] = (
                            gbuf[r, pl.ds(v * _LANES, _LANES)] * _SCALE
                        )

                pltpu.sync_copy(
                    pack_v,
                    out_hbm.at[
                        pl.ds(
                            pl.multiple_of(
                                (base + g * chunk) * _DIM // 128,
                                chunk * _DIM // 128,
                            ),
                            chunk * _DIM // 128,
                        )
                    ],
                )
            return carry

        lax.fori_loop(0, n_chunks // 2, pair_body, 0)

    return k


def kernel(input_ids, table):
    b, h = input_ids.shape
    flat_ids = input_ids.reshape(b * h).astype(jnp.int32)
    table128 = _widen_table_t(table.T)
    packed = _make_gather_kernel(b * h)(flat_ids, table128)
    fin = _transpose_out(packed, b, h)
    return jnp.transpose(fin, (2, 0, 1))


# speedup vs baseline: 1.3770x; 1.3770x over previous
import functools

import jax
import jax.numpy as jnp
from jax import lax
from jax.experimental import pallas as pl
from jax.experimental.pallas import tpu as pltpu
from jax.experimental.pallas import tpu_sc as plsc

_DIM = 64
_SCALE = 8.0
_LANES = 16
_CHUNK = 256
_BB = 128


def _widen_table_t(table_t):
    v = table_t.shape[1]
    block = 2048
    nfull = v // block
    tail = v - nfull * block
    tail_main = (tail // 128) * 128
    tail_rest = tail - tail_main
    grid = nfull + (1 if tail else 0)
    tail_cols = lax.slice(table_t, (0, v - tail_rest), (_DIM, v))

    def body(x_hbm, t_ref, o_ref, xbuf, sem):
        i = pl.program_id(0)

        @pl.when(i < nfull)
        def _full():
            cp = pltpu.make_async_copy(
                x_hbm.at[:, pl.ds(i * block, block)], xbuf, sem
            )
            cp.start()
            cp.wait()

        if tail:

            @pl.when(i == nfull)
            def _tail():
                cp = pltpu.make_async_copy(
                    x_hbm.at[:, pl.ds(nfull * block, tail_main)],
                    xbuf.at[:, pl.ds(0, tail_main)],
                    sem,
                )
                cp.start()
                cp.wait()
                xbuf[:, pl.ds(tail_main, tail_rest)] = t_ref[...]

        eye = jnp.eye(_DIM, dtype=jnp.float32)
        xt = lax.dot_general(
            xbuf[...],
            eye,
            (((0,), (0,)), ((), ())),
            preferred_element_type=jnp.float32,
        )
        o_ref[...] = jnp.concatenate([xt, xt], axis=1)

    return pl.pallas_call(
        body,
        grid=(grid,),
        in_specs=[
            pl.BlockSpec(memory_space=pl.ANY),
            pl.BlockSpec((_DIM, tail_rest), lambda i: (0, 0)),
        ],
        out_specs=pl.BlockSpec((block, 2 * _DIM), lambda i: (i, 0)),
        out_shape=jax.ShapeDtypeStruct((grid * block, 2 * _DIM), jnp.float32),
        scratch_shapes=[
            pltpu.VMEM((_DIM, block), jnp.float32),
            pltpu.SemaphoreType.DMA,
        ],
    )(table_t, tail_cols)


def _transpose_out(packed, b, h):
    j_half = h // 2
    rows_per_blk = _BB * j_half

    def body(x_ref, o_ref):
        x = x_ref[...]
        eye = jnp.eye(128, dtype=jnp.float32)
        for j in range(j_half):
            blk = x[j * 128 : (j + 1) * 128, :]
            t = lax.dot_general(
                blk,
                eye,
                (((0,), (0,)), ((), ())),
                preferred_element_type=jnp.float32,
            )
            o_ref[2 * j, :, :] = t[:_DIM, :]
            o_ref[2 * j + 1, :, :] = t[_DIM:, :]

    return pl.pallas_call(
        body,
        grid=(b // _BB,),
        in_specs=[pl.BlockSpec((rows_per_blk, 128), lambda i: (i, 0))],
        out_specs=pl.BlockSpec((h, _DIM, _BB), lambda i: (0, 0, i)),
        out_shape=jax.ShapeDtypeStruct((h, _DIM, b), jnp.float32),
    )(packed)


def _scatter_rows(batch_flat, h):
    j_half = h // 2
    n_rows = batch_flat * _DIM // 128
    p = jnp.arange(n_rows, dtype=jnp.int32)
    b = p // j_half
    j = p % j_half
    orow = (b // _BB) * (_BB * j_half) + j * _BB + (b % _BB)
    return orow.reshape(n_rows // 128, 128)


@functools.lru_cache(maxsize=None)
def _make_gather_kernel(batch_flat: int, h: int):
    info = plsc.get_sparse_core_info()
    nc, ns = info.num_cores, info.num_subcores
    nw = nc * ns
    assert batch_flat % nw == 0
    b_per_w = batch_flat // nw
    chunk = _CHUNK
    assert b_per_w % (2 * chunk) == 0
    n_chunks = b_per_w // chunk
    vecs_per_row = _DIM // _LANES
    pack_rows = chunk * _DIM // 128

    mesh = plsc.VectorSubcoreMesh(core_axis_name="c", subcore_axis_name="s")

    @functools.partial(
        pl.kernel,
        mesh=mesh,
        out_type=jax.ShapeDtypeStruct((batch_flat * _DIM // 128, 128), jnp.float32),
        scratch_types=[
            pltpu.VMEM((b_per_w,), jnp.int32),
            pltpu.VMEM((2, chunk, 128), jnp.float32),
            pltpu.VMEM((pack_rows, 128), jnp.float32),
            pltpu.VMEM((pack_rows,), jnp.int32),
            pltpu.SemaphoreType.DMA,
            pltpu.SemaphoreType.DMA,
            pltpu.SemaphoreType.DMA,
        ],
    )
    def k(
        ids_hbm,
        table_hbm,
        orows_hbm,
        out_hbm,
        idx_v,
        rows_v,
        pack_v,
        orow_v,
        gsem0,
        gsem1,
        osem,
    ):
        wid = lax.axis_index("s") * nc + lax.axis_index("c")
        base = pl.multiple_of(wid * b_per_w, b_per_w)
        chunk0 = pl.multiple_of(wid * n_chunks, n_chunks)
        gsems = (gsem0, gsem1)

        pltpu.sync_copy(ids_hbm.at[pl.ds(base, b_per_w)], idx_v)
        pltpu.async_copy(
            table_hbm.at[idx_v.at[pl.ds(0, chunk)]], rows_v.at[0], gsem0
        )

        def pair_body(p, carry):
            for b in range(2):
                g = 2 * p + b
                nb = 1 - b

                @pl.when(g + 1 < n_chunks)
                def _start_next():
                    pltpu.async_copy(
                        table_hbm.at[
                            idx_v.at[
                                pl.ds(pl.multiple_of((g + 1) * chunk, chunk), chunk)
                            ]
                        ],
                        rows_v.at[nb],
                        gsems[nb],
                    )

                pltpu.sync_copy(orows_hbm.at[chunk0 + g], orow_v)

                pltpu.make_async_copy(
                    table_hbm.at[idx_v.at[pl.ds(0, chunk)]],
                    rows_v.at[b],
                    gsems[b],
                ).wait()

                gbuf = rows_v.at[b]

                @plsc.parallel_loop(0, chunk, unroll=4)
                def _scale_row(r):
                    half = (r % 2) * _DIM
                    for v in range(vecs_per_row):
                        pack_v[r // 2, pl.ds(half + v * _LANES, _LANES)] = (
                            gbuf[r, pl.ds(v * _LANES, _LANES)] * _SCALE
                        )

                pltpu.async_copy(pack_v, out_hbm.at[orow_v], osem).wait()
            return carry

        lax.fori_loop(0, n_chunks // 2, pair_body, 0)

    return k


def kernel(input_ids, table):
    b, h = input_ids.shape
    flat_ids = input_ids.reshape(b * h).astype(jnp.int32)
    table128 = _widen_table_t(table.T)
    orows = _scatter_rows(b * h, h)
    packed = _make_gather_kernel(b * h, h)(flat_ids, table128, orows)
    fin = _transpose_out(packed, b, h)
    return jnp.transpose(fin, (2, 0, 1))


# speedup vs baseline: 2.1241x; 1.5425x over previous
import functools

import jax
import jax.numpy as jnp
from jax import lax
from jax.experimental import pallas as pl
from jax.experimental.pallas import tpu as pltpu
from jax.experimental.pallas import tpu_sc as plsc

_DIM = 64
_SCALE = 8.0
_LANES = 16
_CHUNK = 256
_BB = 128


def _widen_table_t(table_t):
    v = table_t.shape[1]
    block = 2048
    nfull = v // block
    tail = v - nfull * block
    tail_main = (tail // 128) * 128
    tail_rest = tail - tail_main
    grid = nfull + (1 if tail else 0)
    tail_cols = lax.slice(table_t, (0, v - tail_rest), (_DIM, v))

    def body(x_hbm, t_ref, o_hbm, xbuf, obuf, isem0, isem1, osem0, osem1):
        isems = (isem0, isem1)
        osems = (osem0, osem1)
        eye = jnp.eye(_DIM, dtype=jnp.float32)

        def in_copy(c, slot):
            return pltpu.make_async_copy(
                x_hbm.at[:, pl.ds(pl.multiple_of(c * block, block), block)],
                xbuf.at[slot],
                isems[slot],
            )

        def out_copy(c, slot):
            return pltpu.make_async_copy(
                obuf.at[slot],
                o_hbm.at[pl.ds(pl.multiple_of(c * block, block), block)],
                osems[slot],
            )

        def compute(c, slot):
            xt = lax.dot_general(
                xbuf[slot],
                eye,
                (((0,), (0,)), ((), ())),
                preferred_element_type=jnp.float32,
            )
            obuf[slot] = jnp.concatenate([xt, xt], axis=1)
            out_copy(c, slot).start()

        in_copy(0, 0).start()

        def pair_body(p, carry):
            for b in range(2):
                c = 2 * p + b
                nb = 1 - b

                @pl.when(c + 1 < nfull)
                def _next():
                    in_copy(c + 1, nb).start()

                in_copy(c, b).wait()

                @pl.when(c >= 2)
                def _drain():
                    out_copy(c - 2, b).wait()

                compute(c, b)
            return carry

        lax.fori_loop(0, nfull // 2, pair_body, 0)

        if tail:
            cp = pltpu.make_async_copy(
                x_hbm.at[:, pl.ds(nfull * block, tail_main)],
                xbuf.at[0].at[:, pl.ds(0, tail_main)],
                isems[0],
            )
            cp.start()
            cp.wait()
            xbuf[0, :, pl.ds(tail_main, tail_rest)] = t_ref[...]
            out_copy(nfull - 2, 0).wait()
            compute(nfull, 0)
            out_copy(nfull, 0).wait()
            out_copy(nfull - 1, 1).wait()
        else:
            out_copy(nfull - 2, 0).wait()
            out_copy(nfull - 1, 1).wait()

    return pl.pallas_call(
        body,
        grid=(1,),
        in_specs=[
            pl.BlockSpec(memory_space=pl.ANY),
            pl.BlockSpec((_DIM, tail_rest), lambda i: (0, 0)),
        ],
        out_specs=pl.BlockSpec(memory_space=pl.ANY),
        out_shape=jax.ShapeDtypeStruct((grid * block, 2 * _DIM), jnp.float32),
        scratch_shapes=[
            pltpu.VMEM((2, _DIM, block), jnp.float32),
            pltpu.VMEM((2, block, 2 * _DIM), jnp.float32),
            pltpu.SemaphoreType.DMA,
            pltpu.SemaphoreType.DMA,
            pltpu.SemaphoreType.DMA,
            pltpu.SemaphoreType.DMA,
        ],
    )(table_t, tail_cols)


def _transpose_out(packed, b, h):
    j_half = h // 2
    rows_per_blk = _BB * j_half

    def body(x_ref, o_ref):
        x = x_ref[...]
        eye = jnp.eye(128, dtype=jnp.float32)
        for j in range(j_half):
            blk = x[j * 128 : (j + 1) * 128, :]
            t = lax.dot_general(
                blk,
                eye,
                (((0,), (0,)), ((), ())),
                preferred_element_type=jnp.float32,
            )
            o_ref[2 * j, :, :] = t[:_DIM, :]
            o_ref[2 * j + 1, :, :] = t[_DIM:, :]

    return pl.pallas_call(
        body,
        grid=(b // _BB,),
        in_specs=[pl.BlockSpec((rows_per_blk, 128), lambda i: (i, 0))],
        out_specs=pl.BlockSpec((h, _DIM, _BB), lambda i: (0, 0, i)),
        out_shape=jax.ShapeDtypeStruct((h, _DIM, b), jnp.float32),
    )(packed)


def _scatter_rows(batch_flat, h):
    j_half = h // 2
    n_rows = batch_flat * _DIM // 128
    p = jnp.arange(n_rows, dtype=jnp.int32)
    b = p // j_half
    j = p % j_half
    orow = (b // _BB) * (_BB * j_half) + j * _BB + (b % _BB)
    return orow.reshape(n_rows // 128, 128)


@functools.lru_cache(maxsize=None)
def _make_gather_kernel(batch_flat: int, h: int):
    info = plsc.get_sparse_core_info()
    nc, ns = info.num_cores, info.num_subcores
    nw = nc * ns
    assert batch_flat % nw == 0
    b_per_w = batch_flat // nw
    chunk = _CHUNK
    assert b_per_w % (2 * chunk) == 0
    n_chunks = b_per_w // chunk
    vecs_per_row = _DIM // _LANES
    pack_rows = chunk * _DIM // 128

    mesh = plsc.VectorSubcoreMesh(core_axis_name="c", subcore_axis_name="s")

    @functools.partial(
        pl.kernel,
        mesh=mesh,
        out_type=jax.ShapeDtypeStruct((batch_flat * _DIM // 128, 128), jnp.float32),
        scratch_types=[
            pltpu.VMEM((b_per_w,), jnp.int32),
            pltpu.VMEM((2, chunk, 128), jnp.float32),
            pltpu.VMEM((pack_rows, 128), jnp.float32),
            pltpu.VMEM((pack_rows,), jnp.int32),
            pltpu.SemaphoreType.DMA,
            pltpu.SemaphoreType.DMA,
            pltpu.SemaphoreType.DMA,
        ],
    )
    def k(
        ids_hbm,
        table_hbm,
        orows_hbm,
        out_hbm,
        idx_v,
        rows_v,
        pack_v,
        orow_v,
        gsem0,
        gsem1,
        osem,
    ):
        wid = lax.axis_index("s") * nc + lax.axis_index("c")
        base = pl.multiple_of(wid * b_per_w, b_per_w)
        chunk0 = pl.multiple_of(wid * n_chunks, n_chunks)
        gsems = (gsem0, gsem1)

        pltpu.sync_copy(ids_hbm.at[pl.ds(base, b_per_w)], idx_v)
        pltpu.async_copy(
            table_hbm.at[idx_v.at[pl.ds(0, chunk)]], rows_v.at[0], gsem0
        )

        def pair_body(p, carry):
            for b in range(2):
                g = 2 * p + b
                nb = 1 - b

                @pl.when(g + 1 < n_chunks)
                def _start_next():
                    pltpu.async_copy(
                        table_hbm.at[
                            idx_v.at[
                                pl.ds(pl.multiple_of((g + 1) * chunk, chunk), chunk)
                            ]
                        ],
                        rows_v.at[nb],
                        gsems[nb],
                    )

                pltpu.sync_copy(orows_hbm.at[chunk0 + g], orow_v)

                pltpu.make_async_copy(
                    table_hbm.at[idx_v.at[pl.ds(0, chunk)]],
                    rows_v.at[b],
                    gsems[b],
                ).wait()

                gbuf = rows_v.at[b]

                @plsc.parallel_loop(0, chunk, unroll=4)
                def _scale_row(r):
                    half = (r % 2) * _DIM
                    for v in range(vecs_per_row):
                        pack_v[r // 2, pl.ds(half + v * _LANES, _LANES)] = (
                            gbuf[r, pl.ds(v * _LANES, _LANES)] * _SCALE
                        )

                pltpu.async_copy(pack_v, out_hbm.at[orow_v], osem).wait()
            return carry

        lax.fori_loop(0, n_chunks // 2, pair_body, 0)

    return k


def kernel(input_ids, table):
    b, h = input_ids.shape
    flat_ids = input_ids.reshape(b * h).astype(jnp.int32)
    table128 = _widen_table_t(table.T)
    orows = _scatter_rows(b * h, h)
    packed = _make_gather_kernel(b * h, h)(flat_ids, table128, orows)
    fin = _transpose_out(packed, b, h)
    return jnp.transpose(fin, (2, 0, 1))
